# baseline (speedup 1.0000x reference)
import jax
import jax.numpy as jnp
from jax.experimental import pallas as pl
from jax.experimental.pallas import tpu as pltpu

DIM = 512
K = 2048
NUM_Q = 8
TILE = 512
HALF = TILE // 2
N_TOK = 8 * 1024
N_TILES = N_TOK // TILE


def _rvq_kernel(x_ref, cb_ref, quant_out, idx_out, loss_out,
                res_scr, quant_scr, csum_scr, iota_scr, lo_scr, loss_scr):
    t = pl.program_id(0)
    l = pl.program_id(1)

    @pl.when((t == 0) & (l == 0))
    def _():
        d_all = cb_ref[...]
        c_all = d_all * -0.5
        csum_scr[...] = jnp.sum(c_all * c_all, axis=2)
        hi = d_all.astype(jnp.bfloat16).astype(jnp.float32)
        lo_scr[...] = (d_all - hi).astype(jnp.bfloat16)
        iota_scr[...] = jax.lax.broadcasted_iota(
            jnp.int32, (1, K), 1).astype(jnp.float32)
        loss_scr[0, 0] = 0.0

    @pl.when(l == 0)
    def _():
        res_scr[...] = x_ref[...]
        quant_scr[...] = jnp.zeros((TILE, DIM), jnp.float32)

    d = cb_ref[pl.ds(l, 1)][0]
    lo = lo_scr[pl.ds(l, 1)][0]
    csum = csum_scr[pl.ds(l, 1), :]
    iota = iota_scr[...]
    dn = (((1,), (0,)), ((), ()))

    loss_t = jnp.zeros((), jnp.float32)
    for h in range(2):
        rows = pl.ds(h * HALF, HALF)
        res = res_scr[rows, :]
        rsum = jnp.sum(res * res, axis=-1, keepdims=True)
        ab2 = jax.lax.dot_general(
            res, d, (((1,), (1,)), ((), ())),
            preferred_element_type=jnp.float32)
        dist = (rsum + ab2) + csum

        m = jnp.min(dist, axis=1, keepdims=True)
        idxf = jnp.min(jnp.where(dist == m, iota, float(K)), axis=1,
                       keepdims=True)
        onehot = (iota == idxf).astype(jnp.bfloat16)
        q_lvl = -0.5 * (jax.lax.dot_general(onehot, d, dn,
                                            preferred_element_type=jnp.float32)
                        + jax.lax.dot_general(onehot, lo, dn,
                                              preferred_element_type=jnp.float32))

        diff = res - q_lvl
        res_scr[rows, :] = diff
        quant_scr[rows, :] = quant_scr[rows, :] + q_lvl
        loss_t = loss_t + jnp.sum(m)
        idx_out[0, pl.ds(l, 1), rows] = idxf.astype(jnp.int32).reshape(1, HALF)

    loss_scr[0, 0] += 1.25 * loss_t

    @pl.when(l == NUM_Q - 1)
    def _():
        x = x_ref[...]
        quant_out[...] = x + (quant_scr[...] - x)
        loss_out[...] = jnp.full((1, 1), loss_scr[0, 0], jnp.float32)


def kernel(x, codebooks):
    B, T, D = x.shape
    x2 = x.reshape(B * T, D)
    d = -2.0 * codebooks
    quant, idx, loss = pl.pallas_call(
        _rvq_kernel,
        grid=(N_TILES, NUM_Q),
        in_specs=[
            pl.BlockSpec((TILE, DIM), lambda t, l: (t, 0)),
            pl.BlockSpec((NUM_Q, K, DIM), lambda t, l: (0, 0, 0)),
        ],
        out_specs=[
            pl.BlockSpec((TILE, DIM), lambda t, l: (t, 0)),
            pl.BlockSpec((1, NUM_Q, TILE), lambda t, l: (t, 0, 0)),
            pl.BlockSpec((1, 1), lambda t, l: (0, 0)),
        ],
        out_shape=[
            jax.ShapeDtypeStruct((B * T, D), jnp.float32),
            jax.ShapeDtypeStruct((N_TILES, NUM_Q, TILE), jnp.int32),
            jax.ShapeDtypeStruct((1, 1), jnp.float32),
        ],
        scratch_shapes=[
            pltpu.VMEM((TILE, DIM), jnp.float32),
            pltpu.VMEM((TILE, DIM), jnp.float32),
            pltpu.VMEM((NUM_Q, K), jnp.float32),
            pltpu.VMEM((1, K), jnp.float32),
            pltpu.VMEM((NUM_Q, K, DIM), jnp.bfloat16),
            pltpu.SMEM((1, 1), jnp.float32),
        ],
        compiler_params=pltpu.CompilerParams(
            dimension_semantics=("arbitrary", "arbitrary"),
            vmem_limit_bytes=120 * 1024 * 1024,
        ),
    )(x2, d)
    quantized = quant.reshape(B, T, D)
    indices = idx.transpose(0, 2, 1).reshape(B, T, NUM_Q)
    total_loss = loss[0, 0] / (B * T * D) / NUM_Q
    return quantized, indices, total_loss


# speedup vs baseline: 1.0387x; 1.0387x over previous
import jax
import jax.numpy as jnp
from jax.experimental import pallas as pl
from jax.experimental.pallas import tpu as pltpu

DIM = 512
K = 2048
NUM_Q = 8
TILE = 512
NSUB = 2
SUB = TILE // NSUB
N_TOK = 8 * 1024
N_TILES = N_TOK // TILE


def _rvq_kernel(x_ref, cb_ref, quant_out, idx_out, loss_out,
                res_scr, quant_scr, csum_scr, iota_scr, lo_scr, loss_scr):
    t = pl.program_id(0)
    l = pl.program_id(1)

    @pl.when((t == 0) & (l == 0))
    def _():
        cb_all = cb_ref[...]
        csum_scr[...] = jnp.sum(cb_all * cb_all, axis=2)
        hi = cb_all.astype(jnp.bfloat16).astype(jnp.float32)
        lo_scr[...] = (cb_all - hi).astype(jnp.bfloat16)
        iota_scr[...] = jax.lax.broadcasted_iota(
            jnp.int32, (1, K), 1).astype(jnp.float32)
        loss_scr[0, 0] = 0.0

    @pl.when(l == 0)
    def _():
        res_scr[...] = x_ref[...]
        quant_scr[...] = jnp.zeros((TILE, DIM), jnp.float32)

    cb = cb_ref[pl.ds(l, 1)][0]
    lo = lo_scr[pl.ds(l, 1)][0]
    csum = csum_scr[pl.ds(l, 1), :]
    iota = iota_scr[...]
    dn = (((1,), (0,)), ((), ()))

    loss_t = jnp.zeros((), jnp.float32)
    for h in range(NSUB):
        rows = pl.ds(h * SUB, SUB)
        res = res_scr[rows, :]
        rsum = jnp.sum(res * res, axis=-1, keepdims=True)
        ab = jax.lax.dot_general(
            res, cb, (((1,), (1,)), ((), ())),
            preferred_element_type=jnp.float32)
        dist = (rsum - 2.0 * ab) + csum

        m = jnp.min(dist, axis=1, keepdims=True)
        idxf = jnp.min(jnp.where(dist == m, iota, float(K)), axis=1,
                       keepdims=True)
        onehot = (iota == idxf).astype(jnp.float32)
        q_lvl = (jax.lax.dot_general(onehot, cb, dn,
                                     preferred_element_type=jnp.float32)
                 + jax.lax.dot_general(onehot.astype(jnp.bfloat16), lo, dn,
                                       preferred_element_type=jnp.float32))

        diff = res - q_lvl
        res_scr[rows, :] = diff
        quant_scr[rows, :] = quant_scr[rows, :] + q_lvl
        loss_t = loss_t + jnp.sum(m)
        idx_out[0, pl.ds(l, 1), rows] = idxf.astype(jnp.int32).reshape(1, SUB)

    loss_scr[0, 0] += 1.25 * loss_t

    @pl.when(l == NUM_Q - 1)
    def _():
        x = x_ref[...]
        quant_out[...] = x + (quant_scr[...] - x)
        loss_out[...] = jnp.full((1, 1), loss_scr[0, 0], jnp.float32)


def kernel(x, codebooks):
    B, T, D = x.shape
    x2 = x.reshape(B * T, D)
    quant, idx, loss = pl.pallas_call(
        _rvq_kernel,
        grid=(N_TILES, NUM_Q),
        in_specs=[
            pl.BlockSpec((TILE, DIM), lambda t, l: (t, 0)),
            pl.BlockSpec((NUM_Q, K, DIM), lambda t, l: (0, 0, 0)),
        ],
        out_specs=[
            pl.BlockSpec((TILE, DIM), lambda t, l: (t, 0)),
            pl.BlockSpec((1, NUM_Q, TILE), lambda t, l: (t, 0, 0)),
            pl.BlockSpec((1, 1), lambda t, l: (0, 0)),
        ],
        out_shape=[
            jax.ShapeDtypeStruct((B * T, D), jnp.float32),
            jax.ShapeDtypeStruct((N_TILES, NUM_Q, TILE), jnp.int32),
            jax.ShapeDtypeStruct((1, 1), jnp.float32),
        ],
        scratch_shapes=[
            pltpu.VMEM((TILE, DIM), jnp.float32),
            pltpu.VMEM((TILE, DIM), jnp.float32),
            pltpu.VMEM((NUM_Q, K), jnp.float32),
            pltpu.VMEM((1, K), jnp.float32),
            pltpu.VMEM((NUM_Q, K, DIM), jnp.bfloat16),
            pltpu.SMEM((1, 1), jnp.float32),
        ],
        compiler_params=pltpu.CompilerParams(
            dimension_semantics=("arbitrary", "arbitrary"),
            vmem_limit_bytes=120 * 1024 * 1024,
        ),
    )(x2, codebooks)
    quantized = quant.reshape(B, T, D)
    indices = idx.transpose(0, 2, 1).reshape(B, T, NUM_Q)
    total_loss = loss[0, 0] / (B * T * D) / NUM_Q
    return quantized, indices, total_loss
